# baseline (speedup 1.0000x reference)
import jax
import jax.numpy as jnp
from jax.experimental import pallas as pl


def _gat(x, src, dst, W, a_s, a_d):
    n = x.shape[0]
    h = jnp.einsum('nf,hfo->nho', x, W)
    es = jnp.einsum('nho,ho->nh', h, a_s)
    ed = jnp.einsum('nho,ho->nh', h, a_d)
    e = jax.nn.leaky_relu(es[src] + ed[dst], negative_slope=0.2)
    emax = jax.lax.stop_gradient(jax.ops.segment_max(e, dst, num_segments=n))
    emax = jnp.where(jnp.isfinite(emax), emax, 0.0)
    ee = jnp.exp(e - emax[dst])
    denom = jax.ops.segment_sum(ee, dst, num_segments=n)
    alpha = ee / (denom[dst] + 1e-16)
    out = jax.ops.segment_sum(h[src] * alpha[:, :, None], dst, num_segments=n)
    return jax.nn.elu(out).reshape(n, -1)


def kernel(x_int, x_nh, edge_index_int, edge_index_nh,
           W1i, as1i, ad1i, W1n, as1n, ad1n,
           W2i, as2i, ad2i, W2n, as2n, ad2n,
           W3i, as3i, ad3i, W3n, as3n, ad3n,
           Wd, bd):
    si, di = edge_index_int[0], edge_index_int[1]
    sn, dn = edge_index_nh[0], edge_index_nh[1]
    hi, hn = x_int, x_nh
    p = {
        "W1i": W1i, "as1i": as1i, "ad1i": ad1i,
        "W1n": W1n, "as1n": as1n, "ad1n": ad1n,
        "W2i": W2i, "as2i": as2i, "ad2i": ad2i,
        "W2n": W2n, "as2n": as2n, "ad2n": ad2n,
        "W3i": W3i, "as3i": as3i, "ad3i": ad3i,
        "W3n": W3n, "as3n": as3n, "ad3n": ad3n,
    }
    for l in (1, 2, 3):
        hi = _gat(hi, si, di, p[f"W{l}i"], p[f"as{l}i"], p[f"ad{l}i"])
        hn = _gat(hn, sn, dn, p[f"W{l}n"], p[f"as{l}n"], p[f"ad{l}n"])
    x = jnp.concatenate([hi, hn], axis=1)
    x = jnp.sum(x, axis=0).reshape(1, -1)
    nrm = jnp.maximum(jnp.linalg.norm(x, axis=1, keepdims=True), 1e-12)
    x = x / nrm
    x = x @ Wd + bd
    return jnp.squeeze(x, axis=1)


# speedup vs baseline: 15.8598x; 15.8598x over previous
import functools
import jax
import jax.numpy as jnp
from jax import lax
from jax.experimental import pallas as pl
from jax.experimental.pallas import tpu as pltpu
from jax.experimental.pallas import tpu_sc as plsc

N = 10000
E = 160000
TPC = 16
EPT = E // TPC
RPT = 640
LAST = N - (TPC - 1) * RPT
NB = 5
BN = N // NB


def _make_sc_kernel(fo, K, NSUB):
    W3 = 3 * fo
    P = -(-N // NSUB)
    RT = NSUB // TPC
    FL = 16
    assert EPT % K == 0 and K % 16 == 0
    assert NSUB % TPC == 0 and RT % FL == 0 and N % FL == 0
    mesh = plsc.VectorSubcoreMesh(core_axis_name="c", subcore_axis_name="s")
    out_type = [
        jax.ShapeDtypeStruct((N, W3), jnp.float32),
        jax.ShapeDtypeStruct((N, W3), jnp.float32),
        jax.ShapeDtypeStruct((3 * N,), jnp.float32),
        jax.ShapeDtypeStruct((3 * N,), jnp.float32),
    ]
    scratch = [
        pltpu.VMEM((6 * N,), jnp.float32),
        pltpu.VMEM((K, W3), jnp.float32),
        pltpu.VMEM((K,), jnp.int32),
        pltpu.VMEM((K,), jnp.int32),
        pltpu.VMEM((K,), jnp.int32),
        pltpu.VMEM((3 * K,), jnp.float32),
        pltpu.VMEM((3 * K,), jnp.float32),
        pltpu.VMEM((RPT,), jnp.float32),
        pltpu.VMEM_SHARED((NSUB, W3), jnp.float32),
        pltpu.VMEM_SHARED((N,), jnp.float32),
        pltpu.VMEM_SHARED((N,), jnp.float32),
        pltpu.VMEM_SHARED((N,), jnp.float32),
        pltpu.SemaphoreType.DMA,
    ]

    @functools.partial(pl.kernel, mesh=mesh, out_type=out_type,
                       scratch_types=scratch,
                       compiler_params=pltpu.CompilerParams(
                           needs_layout_passes=False,
                           use_tc_tiling_on_sc=False))
    def sck(src_hbm, dst_hbm, h_lo, h_hi, esed_lo, esed_hi,
            out_lo, out_hi, den_lo, den_hi,
            esed_v, rows, srcb, dstb, dstw, eeb, eedb, zb,
            acc_sh, d0, d1, d2, sem):
        c = lax.axis_index("c")
        w = lax.axis_index("s")
        lanes = lax.iota(jnp.int32, 16)
        zv = jnp.zeros((16,), jnp.float32)

        def run(h_hbm, esed_hbm, out_hbm, den_hbm):
            pltpu.sync_copy(esed_hbm, esed_v)

            def zzb(g, carry):
                zb[pl.ds(g * 16, 16)] = zv
                return carry
            lax.fori_loop(0, RPT // 16, zzb, 0)

            dbase = w * RPT

            @pl.when(w < TPC - 1)
            def _():
                for d in (d0, d1, d2):
                    pltpu.sync_copy(zb, d.at[pl.ds(dbase, RPT)])

            @pl.when(w == TPC - 1)
            def _():
                for d in (d0, d1, d2):
                    pltpu.sync_copy(zb.at[pl.ds(0, LAST)],
                                    d.at[pl.ds(dbase, LAST)])

            def one_pass(p, carry):
                lo = p * NSUB
                for k in range(FL):
                    for t in range(W3 // 16):
                        rows[k, pl.ds(t * 16, 16)] = zv

                def zacc(m, carry2):
                    pltpu.sync_copy(rows.at[pl.ds(0, FL)],
                                    acc_sh.at[pl.ds(w * RT + m * FL, FL)])
                    return carry2
                lax.fori_loop(0, RT // FL, zacc, 0)
                plsc.subcore_barrier()

                def chunk(i, carry2):
                    eb = w * EPT + i * K
                    pltpu.sync_copy(src_hbm.at[pl.ds(eb, K)], srcb)
                    pltpu.sync_copy(dst_hbm.at[pl.ds(eb, K)], dstb)
                    cp = pltpu.async_copy(h_hbm.at[srcb], rows, sem)

                    def grp(g, carry3):
                        s16 = srcb[pl.ds(g * 16, 16)]
                        t16 = dstb[pl.ds(g * 16, 16)]
                        r16 = t16 - lo
                        inr = (r16 >= 0) & (r16 < NSUB)
                        fm = jnp.where(inr, 1.0, 0.0).astype(jnp.float32)
                        dstw[pl.ds(g * 16, 16)] = jnp.where(inr, r16, 0)
                        for j in range(3):
                            es = plsc.load_gather(esed_v, [s16 * 6 + j])
                            ed = plsc.load_gather(esed_v, [t16 * 6 + (3 + j)])
                            e = es + ed
                            ee = jnp.exp(jnp.maximum(e, 0.2 * e))
                            eeb[pl.ds(j * K + g * 16, 16)] = ee * fm

                            @pl.when(p == 0)
                            def _():
                                eedb[pl.ds(j * K + g * 16, 16)] = ee
                        return carry3
                    lax.fori_loop(0, K // 16, grp, 0)
                    cp.wait()

                    def sgrp(g, carry3):
                        k0 = g * 16
                        ridx = k0 + lanes
                        for j in range(3):
                            eev = eeb[pl.ds(j * K + k0, 16)]
                            for t in range(fo // 16):
                                cidx = jnp.full((16,), j * fo + t * 16,
                                                jnp.int32)
                                v = plsc.load_gather(rows, [ridx, cidx])
                                plsc.store_scatter(rows, [ridx, cidx],
                                                   v * eev)
                        return carry3
                    lax.fori_loop(0, K // 16, sgrp, 0)

                    pltpu.sync_copy(rows, acc_sh.at[dstw], add=True)

                    @pl.when(p == 0)
                    def _():
                        pltpu.sync_copy(eedb.at[pl.ds(0, K)],
                                        d0.at[dstb], add=True)
                        pltpu.sync_copy(eedb.at[pl.ds(K, K)],
                                        d1.at[dstb], add=True)
                        pltpu.sync_copy(eedb.at[pl.ds(2 * K, K)],
                                        d2.at[dstb], add=True)
                    return carry2
                lax.fori_loop(0, EPT // K, chunk, 0)
                plsc.subcore_barrier()

                def flsh(m, carry2):
                    rel = w * RT + m * FL
                    ab = lo + rel

                    @pl.when(ab < N)
                    def _():
                        pltpu.sync_copy(acc_sh.at[pl.ds(rel, FL)],
                                        out_hbm.at[pl.ds(ab, FL)])
                    return carry2
                lax.fori_loop(0, RT // FL, flsh, 0)

                @pl.when(p == 0)
                def _():
                    @pl.when(w < TPC - 1)
                    def _():
                        for j, d in enumerate((d0, d1, d2)):
                            pltpu.sync_copy(d.at[pl.ds(dbase, RPT)],
                                            den_hbm.at[pl.ds(j * N + dbase,
                                                             RPT)])

                    @pl.when(w == TPC - 1)
                    def _():
                        for j, d in enumerate((d0, d1, d2)):
                            pltpu.sync_copy(d.at[pl.ds(dbase, LAST)],
                                            den_hbm.at[pl.ds(j * N + dbase,
                                                             LAST)])
                return carry
            lax.fori_loop(0, P, one_pass, 0)

        @pl.when(c == 0)
        def _():
            run(h_lo, esed_lo, out_lo, den_lo)

        @pl.when(c == 1)
        def _():
            run(h_hi, esed_hi, out_hi, den_hi)

    return sck


_SC_KERNELS = {16: _make_sc_kernel(16, 400, 2560),
               32: _make_sc_kernel(32, 400, 2560),
               64: _make_sc_kernel(64, 80, 2560)}


def _full(shape):
    return pl.BlockSpec(shape, lambda i: tuple(0 for _ in shape))


def _rows(shape):
    return pl.BlockSpec(shape, lambda i: (i,) + tuple(0 for _ in shape[1:]))


def _tc_layer1(x, Wf, A):
    fi, W6 = Wf.shape
    W3 = W6 // 2

    def body(x_ref, w_ref, a_ref, hlo_ref, hhi_ref, esed_ref):
        h = jnp.dot(x_ref[...], w_ref[...], preferred_element_type=jnp.float32)
        hlo_ref[...] = h[:, :W3]
        hhi_ref[...] = h[:, W3:]
        esed_ref[...] = jnp.dot(h, a_ref[...], preferred_element_type=jnp.float32)

    return pl.pallas_call(
        body,
        grid=(NB,),
        in_specs=[_rows((BN, fi)), _full((fi, W6)), _full((W6, 12))],
        out_specs=[_rows((BN, W3)), _rows((BN, W3)), _rows((BN, 12))],
        out_shape=[jax.ShapeDtypeStruct((N, W3), jnp.float32),
                   jax.ShapeDtypeStruct((N, W3), jnp.float32),
                   jax.ShapeDtypeStruct((N, 12), jnp.float32)],
    )(x, Wf, A)


def _elu_blocks(al, ah, dn, fp):
    cols = []
    for j in range(3):
        cols.append(al[:, j * fp:(j + 1) * fp] / (dn[:, j:j + 1] + 1e-16))
    for j in range(3):
        cols.append(ah[:, j * fp:(j + 1) * fp] / (dn[:, 3 + j:4 + j] + 1e-16))
    x = jnp.concatenate(cols, axis=1)
    return jnp.where(x > 0, x, jnp.exp(x) - 1.0)


def _tc_layer23(acc_lo, acc_hi, den, Wf, A, fp):
    fi, W6 = Wf.shape
    W3 = W6 // 2
    Wp3 = 3 * fp

    def body(al_ref, ah_ref, dn_ref, w_ref, a_ref, hlo_ref, hhi_ref, esed_ref):
        x = _elu_blocks(al_ref[...], ah_ref[...], dn_ref[...], fp)
        h = jnp.dot(x, w_ref[...], preferred_element_type=jnp.float32)
        hlo_ref[...] = h[:, :W3]
        hhi_ref[...] = h[:, W3:]
        esed_ref[...] = jnp.dot(h, a_ref[...], preferred_element_type=jnp.float32)

    return pl.pallas_call(
        body,
        grid=(NB,),
        in_specs=[_rows((BN, Wp3)), _rows((BN, Wp3)), _rows((BN, 6)),
                  _full((fi, W6)), _full((W6, 12))],
        out_specs=[_rows((BN, W3)), _rows((BN, W3)), _rows((BN, 12))],
        out_shape=[jax.ShapeDtypeStruct((N, W3), jnp.float32),
                   jax.ShapeDtypeStruct((N, W3), jnp.float32),
                   jax.ShapeDtypeStruct((N, 12), jnp.float32)],
    )(acc_lo, acc_hi, den, Wf, A)


def _tc_pool(ai_lo, ai_hi, dni, an_lo, an_hi, dnn):
    fp = 64
    Wp3 = 3 * fp

    def body(ail, aih, dni_ref, anl, anh, dnn_ref, o_ref):
        xi = _elu_blocks(ail[...], aih[...], dni_ref[...], fp)
        xn = _elu_blocks(anl[...], anh[...], dnn_ref[...], fp)
        s = jnp.concatenate([jnp.sum(xi, axis=0, keepdims=True),
                             jnp.sum(xn, axis=0, keepdims=True)], axis=1)

        @pl.when(pl.program_id(0) == 0)
        def _():
            o_ref[...] = s

        @pl.when(pl.program_id(0) != 0)
        def _():
            o_ref[...] = o_ref[...] + s

    return pl.pallas_call(
        body,
        grid=(NB,),
        in_specs=[_rows((BN, Wp3)), _rows((BN, Wp3)), _rows((BN, 6)),
                  _rows((BN, Wp3)), _rows((BN, Wp3)), _rows((BN, 6))],
        out_specs=_full((1, 768)),
        out_shape=jax.ShapeDtypeStruct((1, 768), jnp.float32),
    )(ai_lo, ai_hi, dni, an_lo, an_hi, dnn)


def _tc_head(s, Wd, bd):
    def body(s_ref, wd_ref, bd_ref, o_ref):
        sv = s_ref[...]
        s2 = jnp.sum(sv * sv, axis=1, keepdims=True)
        nrm = jnp.maximum(jnp.sqrt(s2), 1e-12)
        dot = jnp.dot(sv, wd_ref[...], preferred_element_type=jnp.float32)
        o_ref[...] = dot / nrm + bd_ref[...]

    return pl.pallas_call(
        body,
        out_shape=jax.ShapeDtypeStruct((1, 1), jnp.float32),
    )(s, Wd, bd)


def _prep(W, a_s, a_d):
    Hh, fi, fo = W.shape
    Wf = W.transpose(1, 0, 2).reshape(fi, Hh * fo)
    eye = jnp.eye(Hh, dtype=W.dtype)
    As = (eye[:, None, :] * a_s[:, :, None]).reshape(Hh * fo, Hh)
    Ad = (eye[:, None, :] * a_d[:, :, None]).reshape(Hh * fo, Hh)
    A = jnp.concatenate([As[:, :3], Ad[:, :3], As[:, 3:], Ad[:, 3:]], axis=1)
    return Wf, A


def _den6(den_lo, den_hi):
    return jnp.concatenate([den_lo.reshape(3, N),
                            den_hi.reshape(3, N)], axis=0).T


def _branch(x, src, dst, plist):
    (W1, as1, ad1), (W2, as2, ad2), (W3_, as3, ad3) = plist
    fos = [16, 32, 64]
    hlo, hhi, esed = _tc_layer1(x, *_prep(W1, as1, ad1))
    acc_lo, acc_hi, den_lo, den_hi = _SC_KERNELS[16](
        src, dst, hlo, hhi,
        esed[:, :6].reshape(-1), esed[:, 6:].reshape(-1))
    for l, (W, a_s, a_d) in ((2, (W2, as2, ad2)), (3, (W3_, as3, ad3))):
        den = _den6(den_lo, den_hi)
        hlo, hhi, esed = _tc_layer23(acc_lo, acc_hi, den, *_prep(W, a_s, a_d),
                                     fp=fos[l - 2])
        acc_lo, acc_hi, den_lo, den_hi = _SC_KERNELS[fos[l - 1]](
            src, dst, hlo, hhi,
            esed[:, :6].reshape(-1), esed[:, 6:].reshape(-1))
    return acc_lo, acc_hi, _den6(den_lo, den_hi)


def kernel(x_int, x_nh, edge_index_int, edge_index_nh,
           W1i, as1i, ad1i, W1n, as1n, ad1n,
           W2i, as2i, ad2i, W2n, as2n, ad2n,
           W3i, as3i, ad3i, W3n, as3n, ad3n,
           Wd, bd):
    si, di = edge_index_int[0], edge_index_int[1]
    sn, dn = edge_index_nh[0], edge_index_nh[1]
    ai_lo, ai_hi, dni = _branch(
        x_int, si, di,
        [(W1i, as1i, ad1i), (W2i, as2i, ad2i), (W3i, as3i, ad3i)])
    an_lo, an_hi, dnn = _branch(
        x_nh, sn, dn,
        [(W1n, as1n, ad1n), (W2n, as2n, ad2n), (W3n, as3n, ad3n)])
    s = _tc_pool(ai_lo, ai_hi, dni, an_lo, an_hi, dnn)
    out = _tc_head(s, Wd, bd.reshape(1, 1))
    return out.reshape(1)
